# baseline (speedup 1.0000x reference)
import functools

import jax
import jax.numpy as jnp
from jax import lax
from jax.experimental import pallas as pl
from jax.experimental.pallas import tpu as pltpu
from jax.experimental.pallas import tpu_sc as plsc

B = 4096
DIM = 1024
ND = 8

NC = 2
NS = 16
NW = NC * NS
BPW = B // NW
CH = 32
NCH = BPW // CH

T = 256
NT = B // T
MAXU = NT + ND - 1

RR = 32
RL = 128


def _wid():
    return lax.axis_index("s") * NC + lax.axis_index("c")


@functools.lru_cache(maxsize=None)
def _sc_kernels():
    mesh = plsc.VectorSubcoreMesh(core_axis_name="c", subcore_axis_name="s")

    sc_scratch = [
        pltpu.VMEM((NCH, CH), jnp.int32),
        pltpu.VMEM((CH, DIM), jnp.float32),
        pltpu.VMEM((CH, DIM), jnp.float32),
        pltpu.SemaphoreType.DMA,
        pltpu.SemaphoreType.DMA,
        pltpu.SemaphoreType.DMA,
        pltpu.SemaphoreType.DMA,
    ]

    @functools.partial(
        pl.kernel,
        mesh=mesh,
        out_type=jax.ShapeDtypeStruct((B, DIM), jnp.float32),
        scratch_types=sc_scratch,
    )
    def sc_dispatch(mu_hbm, idx_hbm, o_hbm, idx_v, buf0, buf1, si0, si1, so0, so1):
        base = _wid() * BPW
        pltpu.sync_copy(idx_hbm.at[_wid()], idx_v)
        bufs, sin, sout = (buf0, buf1), (si0, si1), (so0, so1)
        cp_in, cp_out = {}, {}
        for ch in range(min(2, NCH)):
            cp_in[ch] = pltpu.async_copy(
                mu_hbm.at[pl.ds(base + ch * CH, CH)], bufs[ch % 2], sin[ch % 2])
        for ch in range(NCH):
            b = ch % 2
            cp_in[ch].wait()
            cp_out[ch] = pltpu.async_copy(bufs[b], o_hbm.at[idx_v.at[ch]], sout[b])
            if ch + 2 < NCH:
                cp_out[ch].wait()
                cp_in[ch + 2] = pltpu.async_copy(
                    mu_hbm.at[pl.ds(base + (ch + 2) * CH, CH)], bufs[b], sin[b])
        for ch in range(max(0, NCH - 2), NCH):
            cp_out[ch].wait()

    @functools.partial(
        pl.kernel,
        mesh=mesh,
        out_type=jax.ShapeDtypeStruct((B, DIM), jnp.float32),
        scratch_types=sc_scratch,
    )
    def sc_combine(ys_hbm, idx_hbm, o_hbm, idx_v, buf0, buf1, si0, si1, so0, so1):
        del buf1, si1, so0, so1
        base = _wid() * BPW
        pltpu.sync_copy(idx_hbm.at[_wid()], idx_v)
        for ch in range(NCH):
            pltpu.async_copy(ys_hbm.at[idx_v.at[ch]], buf0, si0).wait()
            pltpu.sync_copy(buf0, o_hbm.at[pl.ds(base + ch * CH, CH)])

    return sc_dispatch, sc_combine


def _route_body(ids_ref, pos_ref, wk_ref):
    ids = ids_ref[...]
    tri_l = (lax.broadcasted_iota(jnp.int32, (RL, RL), 0) <=
             lax.broadcasted_iota(jnp.int32, (RL, RL), 1)).astype(jnp.float32)
    tri_r = (lax.broadcasted_iota(jnp.int32, (RR, RR), 1) <
             lax.broadcasted_iota(jnp.int32, (RR, RR), 0)).astype(jnp.float32)

    pos = jnp.zeros((RR, RL), jnp.float32)
    off = jnp.float32(0.0)
    offs = []
    for d in range(ND):
        offs.append(off)
        eq = (ids == d).astype(jnp.float32)
        prefix = lax.dot_general(eq, tri_l, (((1,), (0,)), ((), ())),
                                 preferred_element_type=jnp.float32)
        t = jnp.sum(eq, axis=1, keepdims=True)
        before = lax.dot_general(tri_r, t, (((1,), (0,)), ((), ())),
                                 preferred_element_type=jnp.float32)
        rank = before + prefix - eq
        pos = pos + eq * (off + rank)
        off = off + jnp.sum(t)
    offs.append(off)
    pos_ref[...] = pos.astype(jnp.int32)

    ioffs = [o.astype(jnp.int32) for o in offs]
    u = lax.broadcasted_iota(jnp.int32, (1, RL), 1)
    starts_g = []
    start = jnp.int32(0)
    fg_l, ng_l = [], []
    for g in range(ND):
        cnt = ioffs[g + 1] - ioffs[g]
        fg = jnp.minimum(ioffs[g] // T, NT - 1)
        lg = jnp.maximum(ioffs[g + 1] - 1, 0) // T
        ng = jnp.where(cnt > 0, lg - fg + 1, 1)
        starts_g.append(start)
        fg_l.append(fg)
        ng_l.append(ng)
        start = start + ng
    total = start
    uc = jnp.minimum(u, total - 1)
    g_of = jnp.zeros((1, RL), jnp.int32)
    for g in range(ND):
        g_of = g_of + (starts_g[g] <= uc).astype(jnp.int32)
    g_of = g_of - 1
    m_of = jnp.zeros((1, RL), jnp.int32)
    lo = jnp.zeros((1, RL), jnp.int32)
    hi = jnp.zeros((1, RL), jnp.int32)
    for g in range(ND):
        sel = (g_of == g)
        m_g = fg_l[g] + (uc - starts_g[g])
        m_of = jnp.where(sel, m_g, m_of)
        lo = jnp.where(sel, jnp.maximum(ioffs[g], m_g * T), lo)
        hi = jnp.where(sel, jnp.minimum(ioffs[g + 1], (m_g + 1) * T), hi)
    valid = u < total
    lo = jnp.where(valid, lo, 0)
    hi = jnp.where(valid, hi, 0)
    wk_ref[0:1, :] = m_of
    wk_ref[1:2, :] = g_of
    wk_ref[2:3, :] = lo
    wk_ref[3:4, :] = hi


def _routing(ids2):
    return pl.pallas_call(
        _route_body,
        out_shape=[
            jax.ShapeDtypeStruct((RR, RL), jnp.int32),
            jax.ShapeDtypeStruct((4, RL), jnp.int32),
        ],
    )(ids2)


def _mm_body(wk_ref, xs_ref, w_ref, o_ref):
    u = pl.program_id(0)
    up = jnp.maximum(u - 1, 0)
    m = wk_ref[0, u]
    first_m = jnp.logical_or(u == 0, wk_ref[0, up] != m)

    @pl.when(first_m)
    def _():
        o_ref[...] = jnp.zeros_like(o_ref)

    rows = m * T + lax.broadcasted_iota(jnp.int32, (T, 1), 0)
    mask = jnp.logical_and(rows >= wk_ref[2, u], rows < wk_ref[3, u])
    xw = lax.dot_general(
        xs_ref[...], w_ref[0],
        (((1,), (1,)), ((), ())),
        preferred_element_type=jnp.float32,
    )
    o_ref[...] += jnp.where(mask, xw, 0.0)


def _grouped_matmul(wk, xs, W):
    grid_spec = pltpu.PrefetchScalarGridSpec(
        num_scalar_prefetch=1,
        grid=(MAXU,),
        in_specs=[
            pl.BlockSpec((T, DIM), lambda u, wk: (wk[0, u], 0)),
            pl.BlockSpec((1, DIM, DIM), lambda u, wk: (wk[1, u], 0, 0)),
        ],
        out_specs=pl.BlockSpec((T, DIM), lambda u, wk: (wk[0, u], 0)),
    )
    return pl.pallas_call(
        _mm_body,
        grid_spec=grid_spec,
        out_shape=jax.ShapeDtypeStruct((B, DIM), jnp.float32),
    )(wk, xs, W)


ND2 = ND // 2


def _reg_a_body(w_ref, acc_ref, sq_ref):
    i = pl.program_id(0)
    w = w_ref[0]

    @pl.when(i == 0)
    def _():
        acc_ref[...] = w
        sq_ref[...] = w * w

    @pl.when(i != 0)
    def _():
        acc_ref[...] += w
        sq_ref[...] += w * w


def _reg_a(W):
    return pl.pallas_call(
        _reg_a_body,
        grid=(ND2,),
        in_specs=[pl.BlockSpec((1, DIM, DIM), lambda i: (i, 0, 0))],
        out_specs=[
            pl.BlockSpec((DIM, DIM), lambda i: (0, 0)),
            pl.BlockSpec((DIM, DIM), lambda i: (0, 0)),
        ],
        out_shape=[
            jax.ShapeDtypeStruct((DIM, DIM), jnp.float32),
            jax.ShapeDtypeStruct((DIM, DIM), jnp.float32),
        ],
    )(W)


def _reg_b_body(w_ref, acc4_ref, sq4_ref, dep_ref, o_ref, acc_ref, sq_ref):
    del dep_ref
    i = pl.program_id(0)
    w = w_ref[0]

    @pl.when(i == 0)
    def _():
        acc_ref[...] = acc4_ref[...] + w
        sq_ref[...] = sq4_ref[...] + w * w

    @pl.when(i != 0)
    def _():
        acc_ref[...] += w
        sq_ref[...] += w * w

    @pl.when(i == ND2 - 1)
    def _():
        a = acc_ref[...] * (1.0 / ND)
        o_ref[0, 0] = jnp.sum(sq_ref[...]) * (1.0 / (ND * DIM * DIM)) - jnp.sum(
            a * a) * (1.0 / (DIM * DIM))


def _reg_b(W, acc4, sq4, dep):
    return pl.pallas_call(
        _reg_b_body,
        grid=(ND2,),
        in_specs=[
            pl.BlockSpec((1, DIM, DIM), lambda i: (i + ND2, 0, 0)),
            pl.BlockSpec((DIM, DIM), lambda i: (0, 0)),
            pl.BlockSpec((DIM, DIM), lambda i: (0, 0)),
            pl.BlockSpec((8, 128), lambda i: (0, 0)),
        ],
        out_specs=pl.BlockSpec((1, 1), lambda i: (0, 0), memory_space=pltpu.SMEM),
        out_shape=jax.ShapeDtypeStruct((1, 1), jnp.float32),
        scratch_shapes=[
            pltpu.VMEM((DIM, DIM), jnp.float32),
            pltpu.VMEM((DIM, DIM), jnp.float32),
        ],
    )(W, acc4, sq4, dep)


def kernel(mu, domain_ids, W):
    ids2 = domain_ids.astype(jnp.int32).reshape(RR, RL)
    pos, wk = _routing(ids2)
    idx3 = pos.reshape(NW, NCH, CH)

    sc_dispatch, sc_combine = _sc_kernels()
    xs = sc_dispatch(mu, idx3)
    acc4, sq4 = _reg_a(W)
    ys = _grouped_matmul(wk, xs, W)
    out = sc_combine(ys, idx3)
    reg = _reg_b(W, acc4, sq4, ys)
    return out, reg[0, 0]


# speedup vs baseline: 1.0903x; 1.0903x over previous
import functools

import jax
import jax.numpy as jnp
from jax import lax
from jax.experimental import pallas as pl
from jax.experimental.pallas import tpu as pltpu
from jax.experimental.pallas import tpu_sc as plsc

B = 4096
DIM = 1024
ND = 8

NC = 2
NS = 16
NW = NC * NS
BPW = B // NW
CH = 32
NCH = BPW // CH

T = 256
NT = B // T
MAXU = NT + ND - 1

RR = 32
RL = 128


def _wid():
    return lax.axis_index("s") * NC + lax.axis_index("c")


@functools.lru_cache(maxsize=None)
def _sc_kernels():
    mesh = plsc.VectorSubcoreMesh(core_axis_name="c", subcore_axis_name="s")

    sc_scratch = [
        pltpu.VMEM((BPW,), jnp.int32),
        pltpu.VMEM((NCH, CH), jnp.int32),
        pltpu.VMEM((CH, DIM), jnp.float32),
        pltpu.VMEM((CH, DIM), jnp.float32),
        pltpu.SemaphoreType.DMA,
        pltpu.SemaphoreType.DMA,
        pltpu.SemaphoreType.DMA,
        pltpu.SemaphoreType.DMA,
    ]

    def _load_idx(idx_hbm, idx1, idx_v):
        pltpu.sync_copy(idx_hbm.at[_wid()], idx1)
        for k in range(BPW // 16):
            idx_v[k // (CH // 16), pl.ds((k % (CH // 16)) * 16, 16)] = (
                idx1[pl.ds(k * 16, 16)])

    @functools.partial(
        pl.kernel,
        mesh=mesh,
        out_type=jax.ShapeDtypeStruct((B, DIM), jnp.float32),
        scratch_types=sc_scratch,
    )
    def sc_dispatch(mu_hbm, idx_hbm, o_hbm, idx1, idx_v, buf0, buf1,
                    si0, si1, so0, so1):
        base = _wid() * BPW
        _load_idx(idx_hbm, idx1, idx_v)
        bufs, sin, sout = (buf0, buf1), (si0, si1), (so0, so1)
        cp_in, cp_out = {}, {}
        for ch in range(min(2, NCH)):
            cp_in[ch] = pltpu.async_copy(
                mu_hbm.at[pl.ds(base + ch * CH, CH)], bufs[ch % 2], sin[ch % 2])
        for ch in range(NCH):
            b = ch % 2
            cp_in[ch].wait()
            cp_out[ch] = pltpu.async_copy(bufs[b], o_hbm.at[idx_v.at[ch]], sout[b])
            if ch + 2 < NCH:
                cp_out[ch].wait()
                cp_in[ch + 2] = pltpu.async_copy(
                    mu_hbm.at[pl.ds(base + (ch + 2) * CH, CH)], bufs[b], sin[b])
        for ch in range(max(0, NCH - 2), NCH):
            cp_out[ch].wait()

    @functools.partial(
        pl.kernel,
        mesh=mesh,
        out_type=jax.ShapeDtypeStruct((B, DIM), jnp.float32),
        scratch_types=sc_scratch,
    )
    def sc_combine(ys_hbm, idx_hbm, o_hbm, idx1, idx_v, buf0, buf1,
                   si0, si1, so0, so1):
        del buf1, si1, so0, so1
        base = _wid() * BPW
        _load_idx(idx_hbm, idx1, idx_v)
        for ch in range(NCH):
            pltpu.async_copy(ys_hbm.at[idx_v.at[ch]], buf0, si0).wait()
            pltpu.sync_copy(buf0, o_hbm.at[pl.ds(base + ch * CH, CH)])

    return sc_dispatch, sc_combine


def _route_body(ids_ref, pos_ref, wk_ref):
    ids = ids_ref[...]
    tri_l = (lax.broadcasted_iota(jnp.int32, (RL, RL), 0) <=
             lax.broadcasted_iota(jnp.int32, (RL, RL), 1)).astype(jnp.float32)
    tri_r = (lax.broadcasted_iota(jnp.int32, (RR, RR), 1) <
             lax.broadcasted_iota(jnp.int32, (RR, RR), 0)).astype(jnp.float32)

    pos = jnp.zeros((RR, RL), jnp.float32)
    off = jnp.float32(0.0)
    offs = []
    for d in range(ND):
        offs.append(off)
        eq = (ids == d).astype(jnp.float32)
        prefix = lax.dot_general(eq, tri_l, (((1,), (0,)), ((), ())),
                                 preferred_element_type=jnp.float32)
        t = jnp.sum(eq, axis=1, keepdims=True)
        before = lax.dot_general(tri_r, t, (((1,), (0,)), ((), ())),
                                 preferred_element_type=jnp.float32)
        rank = before + prefix - eq
        pos = pos + eq * (off + rank)
        off = off + jnp.sum(t)
    offs.append(off)
    pos_ref[...] = pos.astype(jnp.int32)

    ioffs = [o.astype(jnp.int32) for o in offs]
    u = lax.broadcasted_iota(jnp.int32, (1, RL), 1)
    starts_g = []
    start = jnp.int32(0)
    fg_l, ng_l = [], []
    for g in range(ND):
        cnt = ioffs[g + 1] - ioffs[g]
        fg = jnp.minimum(ioffs[g] // T, NT - 1)
        lg = jnp.maximum(ioffs[g + 1] - 1, 0) // T
        ng = jnp.where(cnt > 0, lg - fg + 1, 1)
        starts_g.append(start)
        fg_l.append(fg)
        ng_l.append(ng)
        start = start + ng
    total = start
    uc = jnp.minimum(u, total - 1)
    g_of = jnp.zeros((1, RL), jnp.int32)
    for g in range(ND):
        g_of = g_of + (starts_g[g] <= uc).astype(jnp.int32)
    g_of = g_of - 1
    m_of = jnp.zeros((1, RL), jnp.int32)
    lo = jnp.zeros((1, RL), jnp.int32)
    hi = jnp.zeros((1, RL), jnp.int32)
    for g in range(ND):
        sel = (g_of == g)
        m_g = fg_l[g] + (uc - starts_g[g])
        m_of = jnp.where(sel, m_g, m_of)
        lo = jnp.where(sel, jnp.maximum(ioffs[g], m_g * T), lo)
        hi = jnp.where(sel, jnp.minimum(ioffs[g + 1], (m_g + 1) * T), hi)
    valid = u < total
    lo = jnp.where(valid, lo, 0)
    hi = jnp.where(valid, hi, 0)
    wk_ref[0:1, :] = m_of
    wk_ref[1:2, :] = g_of
    wk_ref[2:3, :] = lo
    wk_ref[3:4, :] = hi


def _routing(ids2):
    return pl.pallas_call(
        _route_body,
        out_shape=[
            jax.ShapeDtypeStruct((RR, RL), jnp.int32),
            jax.ShapeDtypeStruct((4, RL), jnp.int32),
        ],
    )(ids2)


def _mm_body(wk_ref, xs_ref, w_ref, o_ref):
    u = pl.program_id(0)
    up = jnp.maximum(u - 1, 0)
    m = wk_ref[0, u]
    first_m = jnp.logical_or(u == 0, wk_ref[0, up] != m)

    @pl.when(first_m)
    def _():
        o_ref[...] = jnp.zeros_like(o_ref)

    rows = m * T + lax.broadcasted_iota(jnp.int32, (T, 1), 0)
    mask = jnp.logical_and(rows >= wk_ref[2, u], rows < wk_ref[3, u])
    xw = lax.dot_general(
        xs_ref[...], w_ref[0],
        (((1,), (1,)), ((), ())),
        preferred_element_type=jnp.float32,
    )
    o_ref[...] += jnp.where(mask, xw, 0.0)


def _grouped_matmul(wk, xs, W):
    grid_spec = pltpu.PrefetchScalarGridSpec(
        num_scalar_prefetch=1,
        grid=(MAXU,),
        in_specs=[
            pl.BlockSpec((T, DIM), lambda u, wk: (wk[0, u], 0)),
            pl.BlockSpec((1, DIM, DIM), lambda u, wk: (wk[1, u], 0, 0)),
        ],
        out_specs=pl.BlockSpec((T, DIM), lambda u, wk: (wk[0, u], 0)),
    )
    return pl.pallas_call(
        _mm_body,
        grid_spec=grid_spec,
        out_shape=jax.ShapeDtypeStruct((B, DIM), jnp.float32),
    )(wk, xs, W)


def _reg_body(w_ref, dep_ref, o_ref, acc_ref, sq_ref):
    del dep_ref
    i = pl.program_id(0)
    w = w_ref[0]

    @pl.when(i == 0)
    def _():
        acc_ref[...] = w
        sq_ref[...] = w * w

    @pl.when(i != 0)
    def _():
        acc_ref[...] += w
        sq_ref[...] += w * w

    @pl.when(i == ND - 1)
    def _():
        a = acc_ref[...] * (1.0 / ND)
        o_ref[0, 0] = jnp.sum(sq_ref[...]) * (1.0 / (ND * DIM * DIM)) - jnp.sum(
            a * a) * (1.0 / (DIM * DIM))


def _reg_loss(W, dep):
    return pl.pallas_call(
        _reg_body,
        grid=(ND,),
        in_specs=[
            pl.BlockSpec((1, DIM, DIM), lambda i: (i, 0, 0)),
            pl.BlockSpec((8, 128), lambda i: (0, 0)),
        ],
        out_specs=pl.BlockSpec((1, 1), lambda i: (0, 0), memory_space=pltpu.SMEM),
        out_shape=jax.ShapeDtypeStruct((1, 1), jnp.float32),
        scratch_shapes=[
            pltpu.VMEM((DIM, DIM), jnp.float32),
            pltpu.VMEM((DIM, DIM), jnp.float32),
        ],
    )(W, dep)


def kernel(mu, domain_ids, W):
    ids2 = domain_ids.astype(jnp.int32).reshape(RR, RL)
    pos, wk = _routing(ids2)

    sc_dispatch, sc_combine = _sc_kernels()
    xs = sc_dispatch(mu, pos)
    ys = _grouped_matmul(wk, xs, W)
    out = sc_combine(ys, pos)
    reg = _reg_loss(W, ys)
    return out, reg[0, 0]


# speedup vs baseline: 1.1445x; 1.0497x over previous
import functools

import jax
import jax.numpy as jnp
from jax import lax
from jax.experimental import pallas as pl
from jax.experimental.pallas import tpu as pltpu
from jax.experimental.pallas import tpu_sc as plsc

B = 4096
DIM = 1024
ND = 8

NC = 2
NS = 16
NW = NC * NS
BPW = B // NW
CH = 32
NCH = BPW // CH

T = 512
NT = B // T
MAXU = NT + ND - 1

RR = 32
RL = 128


def _wid():
    return lax.axis_index("s") * NC + lax.axis_index("c")


@functools.lru_cache(maxsize=None)
def _sc_kernels():
    mesh = plsc.VectorSubcoreMesh(core_axis_name="c", subcore_axis_name="s")

    sc_scratch = [
        pltpu.VMEM((BPW,), jnp.int32),
        pltpu.VMEM((NCH, CH), jnp.int32),
        pltpu.VMEM((CH, DIM), jnp.float32),
        pltpu.VMEM((CH, DIM), jnp.float32),
        pltpu.SemaphoreType.DMA,
        pltpu.SemaphoreType.DMA,
        pltpu.SemaphoreType.DMA,
        pltpu.SemaphoreType.DMA,
    ]

    def _load_idx(idx_hbm, idx1, idx_v):
        pltpu.sync_copy(idx_hbm.at[_wid()], idx1)
        for k in range(BPW // 16):
            idx_v[k // (CH // 16), pl.ds((k % (CH // 16)) * 16, 16)] = (
                idx1[pl.ds(k * 16, 16)])

    @functools.partial(
        pl.kernel,
        mesh=mesh,
        out_type=jax.ShapeDtypeStruct((B, DIM), jnp.float32),
        scratch_types=sc_scratch,
    )
    def sc_dispatch(mu_hbm, idx_hbm, o_hbm, idx1, idx_v, buf0, buf1,
                    si0, si1, so0, so1):
        base = _wid() * BPW
        _load_idx(idx_hbm, idx1, idx_v)
        bufs, sin, sout = (buf0, buf1), (si0, si1), (so0, so1)
        cp_in, cp_out = {}, {}
        for ch in range(min(2, NCH)):
            cp_in[ch] = pltpu.async_copy(
                mu_hbm.at[pl.ds(base + ch * CH, CH)], bufs[ch % 2], sin[ch % 2])
        for ch in range(NCH):
            b = ch % 2
            cp_in[ch].wait()
            cp_out[ch] = pltpu.async_copy(bufs[b], o_hbm.at[idx_v.at[ch]], sout[b])
            if ch + 2 < NCH:
                cp_out[ch].wait()
                cp_in[ch + 2] = pltpu.async_copy(
                    mu_hbm.at[pl.ds(base + (ch + 2) * CH, CH)], bufs[b], sin[b])
        for ch in range(max(0, NCH - 2), NCH):
            cp_out[ch].wait()

    @functools.partial(
        pl.kernel,
        mesh=mesh,
        out_type=jax.ShapeDtypeStruct((B, DIM), jnp.float32),
        scratch_types=sc_scratch,
    )
    def sc_combine(ys_hbm, idx_hbm, o_hbm, idx1, idx_v, buf0, buf1,
                   si0, si1, so0, so1):
        del buf1, si1, so0, so1
        base = _wid() * BPW
        _load_idx(idx_hbm, idx1, idx_v)
        for ch in range(NCH):
            pltpu.async_copy(ys_hbm.at[idx_v.at[ch]], buf0, si0).wait()
            pltpu.sync_copy(buf0, o_hbm.at[pl.ds(base + ch * CH, CH)])

    return sc_dispatch, sc_combine


def _route_body(ids_ref, pos_ref, wk_ref):
    ids = ids_ref[...]
    tri_l = (lax.broadcasted_iota(jnp.int32, (RL, RL), 0) <=
             lax.broadcasted_iota(jnp.int32, (RL, RL), 1)).astype(jnp.float32)
    tri_r = (lax.broadcasted_iota(jnp.int32, (RR, RR), 1) <
             lax.broadcasted_iota(jnp.int32, (RR, RR), 0)).astype(jnp.float32)

    pos = jnp.zeros((RR, RL), jnp.float32)
    off = jnp.float32(0.0)
    offs = []
    for d in range(ND):
        offs.append(off)
        eq = (ids == d).astype(jnp.float32)
        prefix = lax.dot_general(eq, tri_l, (((1,), (0,)), ((), ())),
                                 preferred_element_type=jnp.float32)
        t = jnp.sum(eq, axis=1, keepdims=True)
        before = lax.dot_general(tri_r, t, (((1,), (0,)), ((), ())),
                                 preferred_element_type=jnp.float32)
        rank = before + prefix - eq
        pos = pos + eq * (off + rank)
        off = off + jnp.sum(t)
    offs.append(off)
    pos_ref[...] = pos.astype(jnp.int32)

    ioffs = [o.astype(jnp.int32) for o in offs]
    u = lax.broadcasted_iota(jnp.int32, (1, RL), 1)
    starts_g = []
    start = jnp.int32(0)
    fg_l, ng_l = [], []
    for g in range(ND):
        cnt = ioffs[g + 1] - ioffs[g]
        fg = jnp.minimum(ioffs[g] // T, NT - 1)
        lg = jnp.maximum(ioffs[g + 1] - 1, 0) // T
        ng = jnp.where(cnt > 0, lg - fg + 1, 1)
        starts_g.append(start)
        fg_l.append(fg)
        ng_l.append(ng)
        start = start + ng
    total = start
    uc = jnp.minimum(u, total - 1)
    g_of = jnp.zeros((1, RL), jnp.int32)
    for g in range(ND):
        g_of = g_of + (starts_g[g] <= uc).astype(jnp.int32)
    g_of = g_of - 1
    m_of = jnp.zeros((1, RL), jnp.int32)
    lo = jnp.zeros((1, RL), jnp.int32)
    hi = jnp.zeros((1, RL), jnp.int32)
    for g in range(ND):
        sel = (g_of == g)
        m_g = fg_l[g] + (uc - starts_g[g])
        m_of = jnp.where(sel, m_g, m_of)
        lo = jnp.where(sel, jnp.maximum(ioffs[g], m_g * T), lo)
        hi = jnp.where(sel, jnp.minimum(ioffs[g + 1], (m_g + 1) * T), hi)
    valid = u < total
    lo = jnp.where(valid, lo, 0)
    hi = jnp.where(valid, hi, 0)
    wk_ref[0:1, :] = m_of
    wk_ref[1:2, :] = g_of
    wk_ref[2:3, :] = lo
    wk_ref[3:4, :] = hi


def _routing(ids2):
    return pl.pallas_call(
        _route_body,
        out_shape=[
            jax.ShapeDtypeStruct((RR, RL), jnp.int32),
            jax.ShapeDtypeStruct((4, RL), jnp.int32),
        ],
    )(ids2)


def _mm_body(wk_ref, xs_ref, w_ref, o_ref):
    u = pl.program_id(0)
    up = jnp.maximum(u - 1, 0)
    m = wk_ref[0, u]
    first_m = jnp.logical_or(u == 0, wk_ref[0, up] != m)

    @pl.when(first_m)
    def _():
        o_ref[...] = jnp.zeros_like(o_ref)

    rows = m * T + lax.broadcasted_iota(jnp.int32, (T, 1), 0)
    mask = jnp.logical_and(rows >= wk_ref[2, u], rows < wk_ref[3, u])
    xw = lax.dot_general(
        xs_ref[...], w_ref[0],
        (((1,), (1,)), ((), ())),
        preferred_element_type=jnp.float32,
    )
    o_ref[...] += jnp.where(mask, xw, 0.0)


def _grouped_matmul(wk, xs, W):
    grid_spec = pltpu.PrefetchScalarGridSpec(
        num_scalar_prefetch=1,
        grid=(MAXU,),
        in_specs=[
            pl.BlockSpec((T, DIM), lambda u, wk: (wk[0, u], 0)),
            pl.BlockSpec((1, DIM, DIM), lambda u, wk: (wk[1, u], 0, 0)),
        ],
        out_specs=pl.BlockSpec((T, DIM), lambda u, wk: (wk[0, u], 0)),
    )
    return pl.pallas_call(
        _mm_body,
        grid_spec=grid_spec,
        out_shape=jax.ShapeDtypeStruct((B, DIM), jnp.float32),
    )(wk, xs, W)


def _reg_body(w_ref, dep_ref, o_ref, acc_ref, sq_ref):
    del dep_ref
    i = pl.program_id(0)
    w = w_ref[0]

    @pl.when(i == 0)
    def _():
        acc_ref[...] = w
        sq_ref[...] = w * w

    @pl.when(i != 0)
    def _():
        acc_ref[...] += w
        sq_ref[...] += w * w

    @pl.when(i == ND - 1)
    def _():
        a = acc_ref[...] * (1.0 / ND)
        o_ref[0, 0] = jnp.sum(sq_ref[...]) * (1.0 / (ND * DIM * DIM)) - jnp.sum(
            a * a) * (1.0 / (DIM * DIM))


def _reg_loss(W, dep):
    return pl.pallas_call(
        _reg_body,
        grid=(ND,),
        in_specs=[
            pl.BlockSpec((1, DIM, DIM), lambda i: (i, 0, 0)),
            pl.BlockSpec((8, 128), lambda i: (0, 0)),
        ],
        out_specs=pl.BlockSpec((1, 1), lambda i: (0, 0), memory_space=pltpu.SMEM),
        out_shape=jax.ShapeDtypeStruct((1, 1), jnp.float32),
        scratch_shapes=[
            pltpu.VMEM((DIM, DIM), jnp.float32),
            pltpu.VMEM((DIM, DIM), jnp.float32),
        ],
    )(W, dep)


def kernel(mu, domain_ids, W):
    ids2 = domain_ids.astype(jnp.int32).reshape(RR, RL)
    pos, wk = _routing(ids2)

    sc_dispatch, sc_combine = _sc_kernels()
    xs = sc_dispatch(mu, pos)
    ys = _grouped_matmul(wk, xs, W)
    out = sc_combine(ys, pos)
    reg = _reg_loss(W, ys)
    return out, reg[0, 0]
